# Initial kernel scaffold; baseline (speedup 1.0000x reference)
import jax
import jax.numpy as jnp
from jax.experimental import pallas as pl


def kernel(x, edge_index, W1, b1, W2, b2):
    raise NotImplementedError("write your pallas kernel here")


# speedup vs baseline: 4.5555x; 4.5555x over previous
import functools

import jax
import jax.numpy as jnp
from jax import lax
from jax.experimental import pallas as pl
from jax.experimental.pallas import tpu as pltpu
from jax.experimental.pallas import tpu_sc as plsc

NC = 2
NS = 16
NW = NC * NS
CH = 128
DEGW = 16


def _mesh():
    return plsc.VectorSubcoreMesh(
        core_axis_name="c", subcore_axis_name="s",
        num_cores=NC, num_subcores=NS)


def _make_deg_kernel(C, NPAD, RPT):
    @functools.partial(
        pl.kernel,
        out_type=jax.ShapeDtypeStruct((NC, NPAD, DEGW), jnp.float32),
        mesh=_mesh(),
        scratch_types=[
            pltpu.VMEM((C, CH), jnp.int32),
            pltpu.VMEM((CH, DEGW), jnp.float32),
            pltpu.VMEM((RPT, DEGW), jnp.float32),
            pltpu.VMEM_SHARED((NPAD, DEGW), jnp.float32),
        ],
        compiler_params=pltpu.CompilerParams(use_tc_tiling_on_sc=False))
    def deg_kernel(idx_hbm, deg_hbm, idx_v, ones_v, zero_v, deg_s):
        cid = lax.axis_index("c")
        sid = lax.axis_index("s")
        pltpu.sync_copy(idx_hbm.at[cid * NS + sid], idx_v)

        def fill_ones(r, carry):
            ones_v[r] = jnp.full((16,), 1.0, jnp.float32)
            return carry
        lax.fori_loop(0, CH, fill_ones, 0)

        def fill_zero(r, carry):
            zero_v[r] = jnp.zeros((16,), jnp.float32)
            return carry
        lax.fori_loop(0, RPT, fill_zero, 0)
        sl = pl.ds(sid * RPT, RPT)
        pltpu.sync_copy(zero_v, deg_s.at[sl])
        plsc.subcore_barrier()

        def body(j, carry):
            pltpu.sync_copy(ones_v, deg_s.at[idx_v.at[j]], add=True)
            return carry
        lax.fori_loop(0, C, body, 0)
        plsc.subcore_barrier()
        pltpu.sync_copy(deg_s.at[sl], zero_v)
        pltpu.sync_copy(zero_v, deg_hbm.at[cid, sl])

    return deg_kernel


def _make_mp_kernel(C, NPAD, RPT, D):
    assert RPT % CH == 0
    nzb = RPT // CH

    @functools.partial(
        pl.kernel,
        out_type=jax.ShapeDtypeStruct((NC, NPAD, D), jnp.float32),
        mesh=_mesh(),
        scratch_types=[
            pltpu.VMEM((C, CH), jnp.int32),
            pltpu.VMEM((C, CH), jnp.int32),
            pltpu.VMEM((CH, D), jnp.float32),
            pltpu.VMEM_SHARED((NPAD, D), jnp.float32),
            pltpu.SemaphoreType.DMA,
        ],
        compiler_params=pltpu.CompilerParams(use_tc_tiling_on_sc=False))
    def mp_kernel(h_hbm, src_hbm, dst_hbm, out_hbm,
                  src_v, dst_v, rows_v, agg_s, sem):
        cid = lax.axis_index("c")
        sid = lax.axis_index("s")
        wid = cid * NS + sid
        pltpu.sync_copy(src_hbm.at[wid], src_v)
        pltpu.sync_copy(dst_hbm.at[wid], dst_v)

        def zrow(r, carry):
            def zcol(ci, c2):
                rows_v[r, pl.ds(ci * 16, 16)] = jnp.zeros((16,), jnp.float32)
                return c2
            return lax.fori_loop(0, D // 16, zcol, carry)
        lax.fori_loop(0, CH, zrow, 0)

        def zcopy(b, carry):
            pltpu.sync_copy(rows_v, agg_s.at[pl.ds(sid * RPT + b * CH, CH)])
            return carry
        lax.fori_loop(0, nzb, zcopy, 0)
        plsc.subcore_barrier()

        def body(j, carry):
            pltpu.async_copy(h_hbm.at[src_v.at[j]], rows_v, sem).wait()
            pltpu.sync_copy(rows_v, agg_s.at[dst_v.at[j]], add=True)
            return carry
        lax.fori_loop(0, C, body, 0)
        plsc.subcore_barrier()

        def ecopy(b, carry):
            sl = pl.ds(sid * RPT + b * CH, CH)
            pltpu.sync_copy(agg_s.at[sl], rows_v)
            pltpu.sync_copy(rows_v, out_hbm.at[cid, sl])
            return carry
        lax.fori_loop(0, nzb, ecopy, 0)

    return mp_kernel


def _scale_x_block(x_ref, deg_ref, o_ref):
    ns = lax.rsqrt(jnp.maximum(deg_ref[0, :, 0:1], 1.0))
    o_ref[...] = x_ref[...] * ns


def _layer_block(scale_out, agg_ref, deg_ref, w_ref, b_ref, o_ref):
    a = agg_ref[0] + agg_ref[1]
    nd = lax.rsqrt(jnp.maximum(deg_ref[1, :, 0:1], 1.0))
    z = a * nd
    y = jnp.dot(z, w_ref[...], preferred_element_type=jnp.float32) + b_ref[...]
    h = jnp.maximum(y, 0.0)
    if scale_out:
        ns = lax.rsqrt(jnp.maximum(deg_ref[0, :, 0:1], 1.0))
        h = h * ns
    o_ref[...] = h


def _scale_x(x_pad, deg, NPAD, D, RB):
    g = NPAD // RB
    return pl.pallas_call(
        _scale_x_block,
        grid=(g,),
        in_specs=[
            pl.BlockSpec((RB, D), lambda i: (i, 0)),
            pl.BlockSpec((NC, RB, DEGW), lambda i: (0, i, 0)),
        ],
        out_specs=pl.BlockSpec((RB, D), lambda i: (i, 0)),
        out_shape=jax.ShapeDtypeStruct((NPAD, D), jnp.float32),
    )(x_pad, deg)


def _layer(agg, deg, W, b, scale_out, NPAD, D, RB):
    g = NPAD // RB
    return pl.pallas_call(
        functools.partial(_layer_block, scale_out),
        grid=(g,),
        in_specs=[
            pl.BlockSpec((NC, RB, D), lambda i: (0, i, 0)),
            pl.BlockSpec((NC, RB, DEGW), lambda i: (0, i, 0)),
            pl.BlockSpec((D, D), lambda i: (0, 0)),
            pl.BlockSpec((1, D), lambda i: (0, 0)),
        ],
        out_specs=pl.BlockSpec((RB, D), lambda i: (i, 0)),
        out_shape=jax.ShapeDtypeStruct((NPAD, D), jnp.float32),
    )(agg, deg, W, b)


def kernel(x, edge_index, W1, b1, W2, b2):
    N, D = x.shape
    E = edge_index.shape[1]

    RB = 1024
    NPAD = -(-(N + 1) // (NS * CH)) * (NS * CH)
    while NPAD % RB != 0:
        NPAD += NS * CH
    RPT = NPAD // NS

    src = edge_index[0]
    dst = edge_index[1]

    Cw = -(-E // (NW * CH))
    padw = NW * Cw * CH - E
    fillw = jnp.full((padw,), N, jnp.int32)
    srcp = jnp.concatenate([src, fillw]).reshape(NW, Cw, CH)
    dstp = jnp.concatenate([dst, fillw]).reshape(NW, Cw, CH)

    Cd = -(-E // (NS * CH))
    padd = NS * Cd * CH - E
    filld = jnp.full((padd,), N, jnp.int32)
    srcd = jnp.concatenate([src, filld]).reshape(NS, Cd, CH)
    dstd = jnp.concatenate([dst, filld]).reshape(NS, Cd, CH)
    srcdst = jnp.stack([srcd, dstd]).reshape(NC * NS, Cd, CH)

    x_pad = jnp.pad(x, ((0, NPAD - N), (0, 0)))

    deg_k = _make_deg_kernel(Cd, NPAD, RPT)
    mp_k = _make_mp_kernel(Cw, NPAD, RPT, D)

    deg = deg_k(srcdst)
    h0 = _scale_x(x_pad, deg, NPAD, D, RB)
    agg1 = mp_k(h0, srcp, dstp)
    h1 = _layer(agg1, deg, W1, b1.reshape(1, D), True, NPAD, D, RB)
    agg2 = mp_k(h1, srcp, dstp)
    out = _layer(agg2, deg, W2, b2.reshape(1, D), False, NPAD, D, RB)
    return out[:N]
